# Initial kernel scaffold; baseline (speedup 1.0000x reference)
import functools

import jax
import jax.numpy as jnp
from jax import lax
from jax.experimental import pallas as pl
from jax.experimental.pallas import tpu as pltpu
from jax.experimental.pallas import tpu_sc as plsc

N = 100000
D = 128
S = 512
CHUNK = 128
NFULL = N // CHUNK
TAIL = N - NFULL * CHUNK
NC = 2
NS = 16
NW = NC * NS
KBASE = NFULL // NW
NEXTRA = NFULL - KBASE * NW
CW = 16
SPS = S // NS


def _sc_body(x_hbm, batch_hbm, psum_hbm, pcnt_hbm,
             xbuf, idxbuf, tidx, onesbuf, zbuf, zcnt, ssum, scnt):
    cid = lax.axis_index("c")
    sid = lax.axis_index("s")
    w = cid * NS + sid

    zeros16 = jnp.zeros((16,), jnp.float32)
    ones16 = jnp.ones((16,), jnp.float32)

    def fill_z(i, _):
        for j in range(D // 16):
            zbuf[i, pl.ds(j * 16, 16)] = zeros16
        zcnt[i, pl.ds(0, 16)] = zeros16
        return 0

    lax.fori_loop(0, SPS, fill_z, 0)

    def fill_o(i, _):
        onesbuf[i, pl.ds(0, 16)] = ones16
        return 0

    lax.fori_loop(0, CHUNK, fill_o, 0)

    pltpu.sync_copy(zbuf, ssum.at[pl.ds(sid * SPS, SPS), :])
    pltpu.sync_copy(zcnt, scnt.at[pl.ds(sid * SPS, SPS), :])
    plsc.subcore_barrier()

    def process(c):
        pltpu.sync_copy(batch_hbm.at[pl.ds(c * CHUNK, CHUNK)], idxbuf)
        pltpu.sync_copy(x_hbm.at[pl.ds(c * CHUNK, CHUNK), :], xbuf)
        pltpu.sync_copy(xbuf, ssum.at[idxbuf], add=True)
        pltpu.sync_copy(onesbuf, scnt.at[idxbuf], add=True)

    def chunk_body(k, _):
        process(w + NW * k)
        return 0

    lax.fori_loop(0, KBASE, chunk_body, 0)

    @pl.when(w < NEXTRA)
    def _():
        process(w + NW * KBASE)

    @pl.when(w == NW - 1)
    def _():
        base = NFULL * CHUNK
        pltpu.sync_copy(batch_hbm.at[pl.ds(base, TAIL)], tidx)
        pltpu.sync_copy(x_hbm.at[pl.ds(base, TAIL), :],
                        xbuf.at[pl.ds(0, TAIL), :])
        pltpu.sync_copy(xbuf.at[pl.ds(0, TAIL), :], ssum.at[tidx], add=True)
        pltpu.sync_copy(onesbuf.at[pl.ds(0, TAIL), :], scnt.at[tidx],
                        add=True)

    plsc.subcore_barrier()

    pltpu.sync_copy(ssum.at[pl.ds(sid * SPS, SPS), :],
                    psum_hbm.at[cid, pl.ds(sid * SPS, SPS), :])
    pltpu.sync_copy(scnt.at[pl.ds(sid * SPS, SPS), :],
                    pcnt_hbm.at[cid, pl.ds(sid * SPS, SPS), :])


_sc_pool = functools.partial(
    pl.kernel,
    out_type=(jax.ShapeDtypeStruct((NC, S, D), jnp.float32),
              jax.ShapeDtypeStruct((NC, S, CW), jnp.float32)),
    mesh=plsc.VectorSubcoreMesh(core_axis_name="c", subcore_axis_name="s"),
    scratch_types=[
        pltpu.VMEM((CHUNK, D), jnp.float32),
        pltpu.VMEM((CHUNK,), jnp.int32),
        pltpu.VMEM((TAIL,), jnp.int32),
        pltpu.VMEM((CHUNK, CW), jnp.float32),
        pltpu.VMEM((SPS, D), jnp.float32),
        pltpu.VMEM((SPS, CW), jnp.float32),
        pltpu.VMEM_SHARED((S, D), jnp.float32),
        pltpu.VMEM_SHARED((S, CW), jnp.float32),
    ],
)(_sc_body)


def _combine_body(ps_ref, pc_ref, out_ref):
    sums = ps_ref[0] + ps_ref[1]
    cnt = pc_ref[0] + pc_ref[1]
    cnt0 = jnp.maximum(cnt[:, 0:1], 1.0)
    out_ref[...] = sums / cnt0


@jax.jit
def kernel(x, batch):
    psum, pcnt = _sc_pool(x, batch)
    return pl.pallas_call(
        _combine_body,
        out_shape=jax.ShapeDtypeStruct((S, D), jnp.float32),
    )(psum, pcnt)


# speedup vs baseline: 5.6300x; 5.6300x over previous
import functools

import jax
import jax.numpy as jnp
from jax import lax
from jax.experimental import pallas as pl
from jax.experimental.pallas import tpu as pltpu
from jax.experimental.pallas import tpu_sc as plsc

N = 100000
D = 128
S = 512
CHUNK = 128
NFULL = N // CHUNK
TAIL = N - NFULL * CHUNK
NC = 2
NS = 16
NW = NC * NS
KBASE = NFULL // NW
NEXTRA = NFULL - KBASE * NW
SPS = S // NS
L = 16


def _sc_body(x_hbm, batch_hbm, psum_hbm, pcnt_hbm,
             xbuf, idxbuf, zbuf, cnt, ssum):
    cid = lax.axis_index("c")
    sid = lax.axis_index("s")
    w = cid * NS + sid

    zeros16 = jnp.zeros((L,), jnp.float32)
    ones16 = jnp.ones((L,), jnp.float32)

    for i in range(S // L):
        cnt[pl.ds(i * L, L)] = zeros16

    def fill_z(i, _):
        for j in range(D // L):
            zbuf[i, pl.ds(j * L, L)] = zeros16
        return 0

    lax.fori_loop(0, SPS, fill_z, 0)

    pltpu.sync_copy(zbuf, ssum.at[pl.ds(sid * SPS, SPS), :])
    plsc.subcore_barrier()

    def process(c, nrows, xb, idb):
        pltpu.sync_copy(batch_hbm.at[pl.ds(c * CHUNK, nrows)], idb)
        pltpu.sync_copy(x_hbm.at[pl.ds(c * CHUNK, nrows), :], xb)
        pltpu.sync_copy(xb, ssum.at[idb], add=True)
        for k in range(nrows // L):
            idv = idb[pl.ds(k * L, L)]
            plsc.addupdate_scatter(cnt, [idv], ones16)

    def chunk_body(k, _):
        process(w + NW * k, CHUNK, xbuf, idxbuf)
        return 0

    lax.fori_loop(0, KBASE, chunk_body, 0)

    @pl.when(w < NEXTRA)
    def _():
        process(w + NW * KBASE, CHUNK, xbuf, idxbuf)

    @pl.when(w == NW - 1)
    def _():
        base = NFULL * CHUNK
        pltpu.sync_copy(batch_hbm.at[pl.ds(base, TAIL)],
                        idxbuf.at[pl.ds(0, TAIL)])
        for k in range(TAIL // L, CHUNK // L):
            idxbuf[pl.ds(k * L, L)] = jnp.full((L,), S - 1, jnp.int32)
        pltpu.sync_copy(x_hbm.at[pl.ds(base, TAIL), :],
                        xbuf.at[pl.ds(0, TAIL), :])

        def zero_row(i, _):
            for j in range(D // L):
                xbuf[i, pl.ds(j * L, L)] = zeros16
            return 0

        lax.fori_loop(TAIL, CHUNK, zero_row, 0)
        pltpu.sync_copy(xbuf, ssum.at[idxbuf], add=True)
        for k in range(TAIL // L):
            idv = idxbuf[pl.ds(k * L, L)]
            plsc.addupdate_scatter(cnt, [idv], ones16)

    pltpu.sync_copy(cnt, pcnt_hbm.at[w, :])

    plsc.subcore_barrier()

    pltpu.sync_copy(ssum.at[pl.ds(sid * SPS, SPS), :],
                    psum_hbm.at[cid, pl.ds(sid * SPS, SPS), :])


_sc_pool = functools.partial(
    pl.kernel,
    out_type=(jax.ShapeDtypeStruct((NC, S, D), jnp.float32),
              jax.ShapeDtypeStruct((NW, S), jnp.float32)),
    mesh=plsc.VectorSubcoreMesh(core_axis_name="c", subcore_axis_name="s"),
    scratch_types=[
        pltpu.VMEM((CHUNK, D), jnp.float32),
        pltpu.VMEM((CHUNK,), jnp.int32),
        pltpu.VMEM((SPS, D), jnp.float32),
        pltpu.VMEM((S,), jnp.float32),
        pltpu.VMEM_SHARED((S, D), jnp.float32),
    ],
    compiler_params=pltpu.CompilerParams(needs_layout_passes=False),
)(_sc_body)


def _combine_body(ps_ref, pc_ref, out_ref):
    sums = ps_ref[0] + ps_ref[1]
    cnt = jnp.sum(pc_ref[...], axis=0)
    cnt = jnp.maximum(cnt, 1.0)
    out_ref[...] = sums / cnt[:, None]


@jax.jit
def kernel(x, batch):
    psum, pcnt = _sc_pool(x, batch)
    return pl.pallas_call(
        _combine_body,
        out_shape=jax.ShapeDtypeStruct((S, D), jnp.float32),
    )(psum, pcnt)


# speedup vs baseline: 8.3106x; 1.4761x over previous
import functools

import jax
import jax.numpy as jnp
from jax import lax
from jax.experimental import pallas as pl
from jax.experimental.pallas import tpu as pltpu
from jax.experimental.pallas import tpu_sc as plsc

N = 100000
D = 128
S = 512
CHUNK = 128
NFULL = N // CHUNK
TAIL = N - NFULL * CHUNK
NC = 2
NS = 16
NW = NC * NS
KBASE = NFULL // NW
NEXTRA = NFULL - KBASE * NW
SPS = S // NS
L = 16


def _sc_body(x_hbm, batch_hbm, psum_hbm, pcnt_hbm,
             xbuf0, xbuf1, idx0, idx1, zbuf, cnt, ssum,
             semd0, semd1, sems0, sems1):
    cid = lax.axis_index("c")
    sid = lax.axis_index("s")
    w = cid * NS + sid

    xbufs = (xbuf0, xbuf1)
    idxs = (idx0, idx1)
    semds = (semd0, semd1)
    semss = (sems0, sems1)

    zeros16 = jnp.zeros((L,), jnp.float32)
    ones16 = jnp.ones((L,), jnp.float32)

    for i in range(S // L):
        cnt[pl.ds(i * L, L)] = zeros16

    def fill_z(i, _):
        for j in range(D // L):
            zbuf[i, pl.ds(j * L, L)] = zeros16
        return 0

    lax.fori_loop(0, SPS, fill_z, 0)

    pltpu.sync_copy(zbuf, ssum.at[pl.ds(sid * SPS, SPS), :])
    plsc.subcore_barrier()

    def start_dma(c, b):
        pltpu.async_copy(batch_hbm.at[pl.ds(c * CHUNK, CHUNK)], idxs[b],
                         semds[b])
        pltpu.async_copy(x_hbm.at[pl.ds(c * CHUNK, CHUNK), :], xbufs[b],
                         semds[b])

    def wait_dma(b):
        pltpu.make_async_copy(batch_hbm.at[pl.ds(0, CHUNK)], idxs[b],
                              semds[b]).wait()
        pltpu.make_async_copy(x_hbm.at[pl.ds(0, CHUNK), :], xbufs[b],
                              semds[b]).wait()

    def start_scatter(b):
        pltpu.async_copy(xbufs[b], ssum.at[idxs[b]], semss[b], add=True)

    def wait_scatter(b):
        pltpu.make_async_copy(xbufs[b], ssum.at[idxs[b]], semss[b]).wait()

    def histo(b, nv):
        for k in range(nv):
            idv = idxs[b][pl.ds(k * L, L)]
            plsc.addupdate_scatter(cnt, [idv], ones16)

    start_dma(w, 0)
    for k in range(KBASE):
        b = k % 2
        wait_dma(b)
        if k >= 1:
            wait_scatter(1 - b)
        if k + 1 < KBASE:
            start_dma(w + NW * (k + 1), 1 - b)
        else:
            @pl.when(w < NEXTRA)
            def _():
                start_dma(w + NW * KBASE, 1 - b)

            @pl.when(w == NW - 1)
            def _():
                base = NFULL * CHUNK
                pltpu.async_copy(batch_hbm.at[pl.ds(base, TAIL)],
                                 idxs[1 - b].at[pl.ds(0, TAIL)],
                                 semds[1 - b])
                pltpu.async_copy(x_hbm.at[pl.ds(base, TAIL), :],
                                 xbufs[1 - b].at[pl.ds(0, TAIL), :],
                                 semds[1 - b])
        start_scatter(b)
        histo(b, CHUNK // L)

    bl = (KBASE - 1) % 2
    bx = 1 - bl

    @pl.when(w < NEXTRA)
    def _():
        wait_dma(bx)
        wait_scatter(bl)
        start_scatter(bx)
        histo(bx, CHUNK // L)
        wait_scatter(bx)

    @pl.when(w == NW - 1)
    def _():
        base = NFULL * CHUNK
        pltpu.make_async_copy(batch_hbm.at[pl.ds(base, TAIL)],
                              idxs[bx].at[pl.ds(0, TAIL)], semds[bx]).wait()
        pltpu.make_async_copy(x_hbm.at[pl.ds(base, TAIL), :],
                              xbufs[bx].at[pl.ds(0, TAIL), :],
                              semds[bx]).wait()
        for k in range(TAIL // L, CHUNK // L):
            idxs[bx][pl.ds(k * L, L)] = jnp.full((L,), S - 1, jnp.int32)

        def zero_row(i, _):
            for j in range(D // L):
                xbufs[bx][i, pl.ds(j * L, L)] = zeros16
            return 0

        lax.fori_loop(TAIL, CHUNK, zero_row, 0)
        wait_scatter(bl)
        start_scatter(bx)
        histo(bx, TAIL // L)
        wait_scatter(bx)

    @pl.when(jnp.logical_and(w >= NEXTRA, w != NW - 1))
    def _():
        wait_scatter(bl)

    pltpu.sync_copy(cnt, pcnt_hbm.at[w, :])

    plsc.subcore_barrier()

    pltpu.sync_copy(ssum.at[pl.ds(sid * SPS, SPS), :],
                    psum_hbm.at[cid, pl.ds(sid * SPS, SPS), :])


_sc_pool = functools.partial(
    pl.kernel,
    out_type=(jax.ShapeDtypeStruct((NC, S, D), jnp.float32),
              jax.ShapeDtypeStruct((NW, S), jnp.float32)),
    mesh=plsc.VectorSubcoreMesh(core_axis_name="c", subcore_axis_name="s"),
    scratch_types=[
        pltpu.VMEM((CHUNK, D), jnp.float32),
        pltpu.VMEM((CHUNK, D), jnp.float32),
        pltpu.VMEM((CHUNK,), jnp.int32),
        pltpu.VMEM((CHUNK,), jnp.int32),
        pltpu.VMEM((SPS, D), jnp.float32),
        pltpu.VMEM((S,), jnp.float32),
        pltpu.VMEM_SHARED((S, D), jnp.float32),
        pltpu.SemaphoreType.DMA,
        pltpu.SemaphoreType.DMA,
        pltpu.SemaphoreType.DMA,
        pltpu.SemaphoreType.DMA,
    ],
    compiler_params=pltpu.CompilerParams(needs_layout_passes=False),
)(_sc_body)


def _combine_body(ps_ref, pc_ref, out_ref):
    sums = ps_ref[0] + ps_ref[1]
    cnt = jnp.sum(pc_ref[...], axis=0)
    cnt = jnp.maximum(cnt, 1.0)
    out_ref[...] = sums / cnt[:, None]


@jax.jit
def kernel(x, batch):
    psum, pcnt = _sc_pool(x, batch)
    return pl.pallas_call(
        _combine_body,
        out_shape=jax.ShapeDtypeStruct((S, D), jnp.float32),
    )(psum, pcnt)


# speedup vs baseline: 8.4670x; 1.0188x over previous
import functools

import jax
import jax.numpy as jnp
from jax import lax
from jax.experimental import pallas as pl
from jax.experimental.pallas import tpu as pltpu
from jax.experimental.pallas import tpu_sc as plsc

N = 100000
D = 128
S = 512
U = 128
CHUNK = 256
NFULL = N // CHUNK
TAILA = 128
TAILB = N - NFULL * CHUNK - TAILA
NC = 2
NS = 16
NW = NC * NS
KBASE = NFULL // NW
NEXTRA = NFULL - KBASE * NW
SPS = S // NS
L = 16


def _sc_body(x_hbm, batch_hbm, psum_hbm, pcnt_hbm,
             xbuf0, xbuf1, idx0, idx1, zbuf, cnt, ssum,
             semd0, semd1, sems0, sems1):
    cid = lax.axis_index("c")
    sid = lax.axis_index("s")
    w = cid * NS + sid

    xbufs = (xbuf0, xbuf1)
    idxs = (idx0, idx1)
    semds = (semd0, semd1)
    semss = (sems0, sems1)

    zeros16 = jnp.zeros((L,), jnp.float32)
    ones16 = jnp.ones((L,), jnp.float32)

    for i in range(S // L):
        cnt[pl.ds(i * L, L)] = zeros16

    def fill_z(i, _):
        for j in range(D // L):
            zbuf[i, pl.ds(j * L, L)] = zeros16
        return 0

    lax.fori_loop(0, SPS, fill_z, 0)

    pltpu.sync_copy(zbuf, ssum.at[pl.ds(sid * SPS, SPS), :])
    plsc.subcore_barrier()

    def start_dma(c, b):
        pltpu.async_copy(batch_hbm.at[pl.ds(c * CHUNK, U)],
                         idxs[b].at[0], semds[b])
        pltpu.async_copy(batch_hbm.at[pl.ds(c * CHUNK + U, U)],
                         idxs[b].at[1], semds[b])
        pltpu.async_copy(x_hbm.at[pl.ds(c * CHUNK, CHUNK), :], xbufs[b],
                         semds[b])

    def wait_dma(b):
        pltpu.make_async_copy(batch_hbm.at[pl.ds(0, U)], idxs[b].at[0],
                              semds[b]).wait()
        pltpu.make_async_copy(batch_hbm.at[pl.ds(0, U)], idxs[b].at[1],
                              semds[b]).wait()
        pltpu.make_async_copy(x_hbm.at[pl.ds(0, CHUNK), :], xbufs[b],
                              semds[b]).wait()

    def start_scatter(b):
        pltpu.async_copy(xbufs[b].at[pl.ds(0, U), :],
                         ssum.at[idxs[b].at[0]], semss[b], add=True)
        pltpu.async_copy(xbufs[b].at[pl.ds(U, U), :],
                         ssum.at[idxs[b].at[1]], semss[b], add=True)

    def wait_scatter(b):
        pltpu.make_async_copy(xbufs[b].at[pl.ds(0, U), :],
                              ssum.at[idxs[b].at[0]], semss[b]).wait()
        pltpu.make_async_copy(xbufs[b].at[pl.ds(U, U), :],
                              ssum.at[idxs[b].at[1]], semss[b]).wait()

    def histo(b, row, nv):
        for k in range(nv):
            idv = idxs[b][row, pl.ds(k * L, L)]
            plsc.addupdate_scatter(cnt, [idv], ones16)

    start_dma(w, 0)
    for k in range(KBASE):
        b = k % 2
        wait_dma(b)
        if k >= 1:
            wait_scatter(1 - b)
        if k + 1 < KBASE:
            start_dma(w + NW * (k + 1), 1 - b)
        else:
            @pl.when(w < NEXTRA)
            def _():
                start_dma(w + NW * KBASE, 1 - b)

            @pl.when(w == NW - 1)
            def _():
                base = NFULL * CHUNK
                bb = 1 - b
                pltpu.async_copy(batch_hbm.at[pl.ds(base, TAILA)],
                                 idxs[bb].at[0], semds[bb])
                pltpu.async_copy(batch_hbm.at[pl.ds(base + TAILA, TAILB)],
                                 idxs[bb].at[1, pl.ds(0, TAILB)], semds[bb])
                pltpu.async_copy(x_hbm.at[pl.ds(base, TAILA + TAILB), :],
                                 xbufs[bb].at[pl.ds(0, TAILA + TAILB), :],
                                 semds[bb])
        start_scatter(b)
        histo(b, 0, U // L)
        histo(b, 1, U // L)

    bl = (KBASE - 1) % 2
    bx = 1 - bl

    @pl.when(w < NEXTRA)
    def _():
        wait_dma(bx)
        wait_scatter(bl)
        start_scatter(bx)
        histo(bx, 0, U // L)
        histo(bx, 1, U // L)
        wait_scatter(bx)

    @pl.when(w == NW - 1)
    def _():
        base = NFULL * CHUNK
        pltpu.make_async_copy(batch_hbm.at[pl.ds(base, TAILA)],
                              idxs[bx].at[0], semds[bx]).wait()
        pltpu.make_async_copy(batch_hbm.at[pl.ds(base, TAILB)],
                              idxs[bx].at[1, pl.ds(0, TAILB)],
                              semds[bx]).wait()
        pltpu.make_async_copy(x_hbm.at[pl.ds(base, TAILA + TAILB), :],
                              xbufs[bx].at[pl.ds(0, TAILA + TAILB), :],
                              semds[bx]).wait()
        for k in range(TAILB // L, U // L):
            idxs[bx][1, pl.ds(k * L, L)] = jnp.full((L,), S - 1, jnp.int32)

        def zero_row(i, _):
            for j in range(D // L):
                xbufs[bx][i, pl.ds(j * L, L)] = zeros16
            return 0

        lax.fori_loop(TAILA + TAILB, CHUNK, zero_row, 0)
        wait_scatter(bl)
        start_scatter(bx)
        histo(bx, 0, U // L)
        histo(bx, 1, TAILB // L)
        wait_scatter(bx)

    @pl.when(jnp.logical_and(w >= NEXTRA, w != NW - 1))
    def _():
        wait_scatter(bl)

    pltpu.sync_copy(cnt, pcnt_hbm.at[w, :])

    plsc.subcore_barrier()

    pltpu.sync_copy(ssum.at[pl.ds(sid * SPS, SPS), :],
                    psum_hbm.at[cid, pl.ds(sid * SPS, SPS), :])


_sc_pool = functools.partial(
    pl.kernel,
    out_type=(jax.ShapeDtypeStruct((NC, S, D), jnp.float32),
              jax.ShapeDtypeStruct((NW, S), jnp.float32)),
    mesh=plsc.VectorSubcoreMesh(core_axis_name="c", subcore_axis_name="s"),
    scratch_types=[
        pltpu.VMEM((CHUNK, D), jnp.float32),
        pltpu.VMEM((CHUNK, D), jnp.float32),
        pltpu.VMEM((2, U), jnp.int32),
        pltpu.VMEM((2, U), jnp.int32),
        pltpu.VMEM((SPS, D), jnp.float32),
        pltpu.VMEM((S,), jnp.float32),
        pltpu.VMEM_SHARED((S, D), jnp.float32),
        pltpu.SemaphoreType.DMA,
        pltpu.SemaphoreType.DMA,
        pltpu.SemaphoreType.DMA,
        pltpu.SemaphoreType.DMA,
    ],
    compiler_params=pltpu.CompilerParams(needs_layout_passes=False),
)(_sc_body)


def _combine_body(ps_ref, pc_ref, out_ref):
    sums = ps_ref[0] + ps_ref[1]
    cnt = jnp.sum(pc_ref[...], axis=0)
    cnt = jnp.maximum(cnt, 1.0)
    out_ref[...] = sums / cnt[:, None]


@jax.jit
def kernel(x, batch):
    psum, pcnt = _sc_pool(x, batch)
    return pl.pallas_call(
        _combine_body,
        out_shape=jax.ShapeDtypeStruct((S, D), jnp.float32),
    )(psum, pcnt)


# speedup vs baseline: 9.0645x; 1.0706x over previous
import functools

import jax
import jax.numpy as jnp
from jax import lax
from jax.experimental import pallas as pl
from jax.experimental.pallas import tpu as pltpu
from jax.experimental.pallas import tpu_sc as plsc

N = 100000
D = 128
S = 512
U = 128
CHUNK = 256
NFULL = N // CHUNK
TAILA = 128
TAILB = N - NFULL * CHUNK - TAILA
NC = 2
NS = 16
NW = NC * NS
KBASE = NFULL // NW
NEXTRA = NFULL - KBASE * NW
NL = KBASE + 1
NBUF = 3
SPS = S // NS
L = 16


def _sc_body(x_hbm, batch_hbm, psum_hbm, pcnt_hbm,
             xbuf0, xbuf1, xbuf2, idx0, idx1, idx2, zbuf, cnt, ssum,
             semd0, semd1, semd2, sems0, sems1, sems2):
    cid = lax.axis_index("c")
    sid = lax.axis_index("s")
    w = cid * NS + sid

    xbufs = (xbuf0, xbuf1, xbuf2)
    idxs = (idx0, idx1, idx2)
    semds = (semd0, semd1, semd2)
    semss = (sems0, sems1, sems2)

    zeros16 = jnp.zeros((L,), jnp.float32)
    ones16 = jnp.ones((L,), jnp.float32)

    for i in range(S // L):
        cnt[pl.ds(i * L, L)] = zeros16

    def fill_z(i, _):
        for j in range(D // L):
            zbuf[i, pl.ds(j * L, L)] = zeros16
        return 0

    lax.fori_loop(0, SPS, fill_z, 0)

    pltpu.sync_copy(zbuf, ssum.at[pl.ds(sid * SPS, SPS), :])
    plsc.subcore_barrier()

    def start_dma(c, b):
        pltpu.async_copy(batch_hbm.at[pl.ds(c * CHUNK, U)],
                         idxs[b].at[0], semds[b])
        pltpu.async_copy(batch_hbm.at[pl.ds(c * CHUNK + U, U)],
                         idxs[b].at[1], semds[b])
        pltpu.async_copy(x_hbm.at[pl.ds(c * CHUNK, CHUNK), :], xbufs[b],
                         semds[b])

    def wait_dma(b):
        pltpu.make_async_copy(batch_hbm.at[pl.ds(0, U)], idxs[b].at[0],
                              semds[b]).wait()
        pltpu.make_async_copy(batch_hbm.at[pl.ds(0, U)], idxs[b].at[1],
                              semds[b]).wait()
        pltpu.make_async_copy(x_hbm.at[pl.ds(0, CHUNK), :], xbufs[b],
                              semds[b]).wait()

    def start_tail_dma(b):
        base = NFULL * CHUNK
        pltpu.async_copy(batch_hbm.at[pl.ds(base, TAILA)],
                         idxs[b].at[0], semds[b])
        pltpu.async_copy(batch_hbm.at[pl.ds(base + TAILA, TAILB)],
                         idxs[b].at[1, pl.ds(0, TAILB)], semds[b])
        pltpu.async_copy(x_hbm.at[pl.ds(base, TAILA + TAILB), :],
                         xbufs[b].at[pl.ds(0, TAILA + TAILB), :], semds[b])

    def wait_tail_dma(b):
        base = NFULL * CHUNK
        pltpu.make_async_copy(batch_hbm.at[pl.ds(base, TAILA)],
                              idxs[b].at[0], semds[b]).wait()
        pltpu.make_async_copy(batch_hbm.at[pl.ds(base, TAILB)],
                              idxs[b].at[1, pl.ds(0, TAILB)],
                              semds[b]).wait()
        pltpu.make_async_copy(x_hbm.at[pl.ds(base, TAILA + TAILB), :],
                              xbufs[b].at[pl.ds(0, TAILA + TAILB), :],
                              semds[b]).wait()

    def start_scatter(b):
        pltpu.async_copy(xbufs[b].at[pl.ds(0, U), :],
                         ssum.at[idxs[b].at[0]], semss[b], add=True)
        pltpu.async_copy(xbufs[b].at[pl.ds(U, U), :],
                         ssum.at[idxs[b].at[1]], semss[b], add=True)

    def wait_scatter(b):
        pltpu.make_async_copy(xbufs[b].at[pl.ds(0, U), :],
                              ssum.at[idxs[b].at[0]], semss[b]).wait()
        pltpu.make_async_copy(xbufs[b].at[pl.ds(U, U), :],
                              ssum.at[idxs[b].at[1]], semss[b]).wait()

    def histo(b, row, nv):
        for k in range(nv):
            idv = idxs[b][row, pl.ds(k * L, L)]
            plsc.addupdate_scatter(cnt, [idv], ones16)

    is_extra = w < NEXTRA
    is_tailw = w == NW - 1

    def start_logical(i, b):
        if i < KBASE:
            start_dma(w + NW * i, b)
        else:
            @pl.when(is_extra)
            def _():
                start_dma(w + NW * KBASE, b)

            @pl.when(is_tailw)
            def _():
                start_tail_dma(b)

    def process_logical(i, b):
        if i < KBASE:
            wait_dma(b)
            start_scatter(b)
            histo(b, 0, U // L)
            histo(b, 1, U // L)
        else:
            @pl.when(is_extra)
            def _():
                wait_dma(b)
                start_scatter(b)
                histo(b, 0, U // L)
                histo(b, 1, U // L)

            @pl.when(is_tailw)
            def _():
                wait_tail_dma(b)
                for k in range(TAILB // L, U // L):
                    idxs[b][1, pl.ds(k * L, L)] = jnp.full((L,), S - 1,
                                                           jnp.int32)

                def zero_row(r, _):
                    for j in range(D // L):
                        xbufs[b][r, pl.ds(j * L, L)] = zeros16
                    return 0

                lax.fori_loop(TAILA + TAILB, CHUNK, zero_row, 0)
                start_scatter(b)
                histo(b, 0, U // L)
                histo(b, 1, TAILB // L)

    def wait_scatter_logical(i, b):
        if i < KBASE:
            wait_scatter(b)
        else:
            @pl.when(jnp.logical_or(is_extra, is_tailw))
            def _():
                wait_scatter(b)

    start_logical(0, 0)
    start_logical(1, 1)
    for i in range(NL):
        b = i % NBUF
        if i + 2 < NL:
            if i + 2 >= NBUF:
                wait_scatter_logical(i + 2 - NBUF, (i + 2) % NBUF)
            start_logical(i + 2, (i + 2) % NBUF)
        process_logical(i, b)
    for i in range(max(0, NL - NBUF), NL):
        wait_scatter_logical(i, i % NBUF)

    pltpu.sync_copy(cnt, pcnt_hbm.at[w, :])

    plsc.subcore_barrier()

    pltpu.sync_copy(ssum.at[pl.ds(sid * SPS, SPS), :],
                    psum_hbm.at[cid, pl.ds(sid * SPS, SPS), :])


_sc_pool = functools.partial(
    pl.kernel,
    out_type=(jax.ShapeDtypeStruct((NC, S, D), jnp.float32),
              jax.ShapeDtypeStruct((NW, S), jnp.float32)),
    mesh=plsc.VectorSubcoreMesh(core_axis_name="c", subcore_axis_name="s"),
    scratch_types=[
        pltpu.VMEM((CHUNK, D), jnp.float32),
        pltpu.VMEM((CHUNK, D), jnp.float32),
        pltpu.VMEM((CHUNK, D), jnp.float32),
        pltpu.VMEM((2, U), jnp.int32),
        pltpu.VMEM((2, U), jnp.int32),
        pltpu.VMEM((2, U), jnp.int32),
        pltpu.VMEM((SPS, D), jnp.float32),
        pltpu.VMEM((S,), jnp.float32),
        pltpu.VMEM_SHARED((S, D), jnp.float32),
        pltpu.SemaphoreType.DMA,
        pltpu.SemaphoreType.DMA,
        pltpu.SemaphoreType.DMA,
        pltpu.SemaphoreType.DMA,
        pltpu.SemaphoreType.DMA,
        pltpu.SemaphoreType.DMA,
    ],
    compiler_params=pltpu.CompilerParams(needs_layout_passes=False),
)(_sc_body)


def _combine_body(ps_ref, pc_ref, out_ref):
    sums = ps_ref[0] + ps_ref[1]
    cnt = jnp.sum(pc_ref[...], axis=0)
    cnt = jnp.maximum(cnt, 1.0)
    out_ref[...] = sums / cnt[:, None]


@jax.jit
def kernel(x, batch):
    psum, pcnt = _sc_pool(x, batch)
    return pl.pallas_call(
        _combine_body,
        out_shape=jax.ShapeDtypeStruct((S, D), jnp.float32),
    )(psum, pcnt)
